# baseline (speedup 1.0000x reference)
import jax
import jax.numpy as jnp
from jax import lax
from jax.experimental import pallas as pl
from jax.experimental.pallas import tpu as pltpu

B = 4
C = 384
H = W = 224
HW = H * W
K_KEEP = C // 2
HB = 16
NH = H // HB


def _sum_body(x_ref, out_ref):
    h = pl.program_id(1)
    partial = jnp.sum(x_ref[0], axis=(0, 1))

    @pl.when(h == 0)
    def _():
        out_ref[0, 0] = partial

    @pl.when(h != 0)
    def _():
        out_ref[0, 0] += partial


def _mask_body(y2_ref, out_ref):
    y2 = y2_ref[...]

    a = y2[:, :, None]
    b = y2[:, None, :]
    ii = lax.broadcasted_iota(jnp.int32, (B, C, C), 1)
    jj = lax.broadcasted_iota(jnp.int32, (B, C, C), 2)
    beats = jnp.logical_or(b > a, jnp.logical_and(b == a, jj < ii))
    rank = jnp.sum(beats.astype(jnp.float32), axis=2)
    out_ref[...] = jnp.where(rank < K_KEEP, y2, 0.0)


def _scale_body(x_ref, y3_ref, out_ref):
    out_ref[0] = x_ref[0] * y3_ref[0, 0, 0][None, None, :]


@jax.jit
def kernel(x, conv_w):
    xt = jnp.transpose(x, (0, 2, 3, 1))

    sums3 = pl.pallas_call(
        _sum_body,
        grid=(B, NH),
        in_specs=[pl.BlockSpec((1, HB, W, C), lambda s, h: (s, h, 0, 0))],
        out_specs=pl.BlockSpec((1, 1, C), lambda s, h: (s, 0, 0)),
        out_shape=jax.ShapeDtypeStruct((B, 1, C), jnp.float32),
    )(xt)
    sums = sums3.reshape(B, C)

    y = sums / HW
    y2 = jax.lax.conv_general_dilated(
        y[:, None, :], conv_w,
        window_strides=(1,), padding=[(1, 1)],
        dimension_numbers=('NCH', 'OIH', 'NCH'))[:, 0, :]
    y2 = jax.nn.sigmoid(y2)

    y3 = pl.pallas_call(
        _mask_body,
        in_specs=[pl.BlockSpec((B, C), lambda: (0, 0))],
        out_shape=jax.ShapeDtypeStruct((B, C), jnp.float32),
    )(y2)

    y3r = y3.reshape(B, 1, 1, C)
    out_t = pl.pallas_call(
        _scale_body,
        grid=(B, NH),
        in_specs=[
            pl.BlockSpec((1, HB, W, C), lambda s, h: (s, h, 0, 0)),
            pl.BlockSpec((1, 1, 1, C), lambda s, h: (s, 0, 0, 0)),
        ],
        out_specs=pl.BlockSpec((1, HB, W, C), lambda s, h: (s, h, 0, 0)),
        out_shape=jax.ShapeDtypeStruct((B, H, W, C), jnp.float32),
    )(xt, y3r)

    return jnp.transpose(out_t, (0, 3, 1, 2))
